# Initial kernel scaffold; baseline (speedup 1.0000x reference)
import jax
import jax.numpy as jnp
from jax.experimental import pallas as pl


def kernel(x, g_id, embedding):
    raise NotImplementedError("write your pallas kernel here")


# speedup vs baseline: 1.0358x; 1.0358x over previous
import functools

import jax
import jax.numpy as jnp
from jax.experimental import pallas as pl

GRID_NUNQ = 384
HIDDEN = 128
HIST = 200
TB = 32


def _body(gid_ref, x_ref, emb_ref, o_ref):
    ids = gid_ref[0]
    one_hot = (
        jax.lax.broadcasted_iota(jnp.int32, (GRID_NUNQ, TB), 0) == ids
    ).astype(jnp.float32)
    emb_tile = jax.lax.dot_general(
        one_hot,
        emb_ref[...],
        (((0,), (0,)), ((), ())),
        preferred_element_type=jnp.float32,
    )
    o_ref[...] = x_ref[...] + emb_tile[:, None, :]


@jax.jit
def kernel(x, g_id, embedding):
    batch = x.shape[0]
    num_blocks = batch // TB
    gid3 = g_id.astype(jnp.int32).reshape(num_blocks, 1, TB)
    return pl.pallas_call(
        _body,
        grid=(num_blocks,),
        in_specs=[
            pl.BlockSpec((1, 1, TB), lambda i: (i, 0, 0)),
            pl.BlockSpec((TB, HIST, HIDDEN), lambda i: (i, 0, 0)),
            pl.BlockSpec((GRID_NUNQ, HIDDEN), lambda i: (0, 0)),
        ],
        out_specs=pl.BlockSpec((TB, HIST, HIDDEN), lambda i: (i, 0, 0)),
        out_shape=jax.ShapeDtypeStruct((batch, HIST, HIDDEN), jnp.float32),
    )(gid3, x, embedding)


# speedup vs baseline: 1.0595x; 1.0229x over previous
import functools

import jax
import jax.numpy as jnp
from jax.experimental import pallas as pl
from jax.experimental.pallas import tpu as pltpu

GRID_NUNQ = 384
HIDDEN = 128
HIST = 200
TB = 64


def _body(gid_ref, x_ref, emb_ref, o_ref):
    ids = gid_ref[0]
    one_hot = (
        jax.lax.broadcasted_iota(jnp.int32, (GRID_NUNQ, TB), 0) == ids
    ).astype(jnp.float32)
    emb_tile = jax.lax.dot_general(
        one_hot,
        emb_ref[...],
        (((0,), (0,)), ((), ())),
        preferred_element_type=jnp.float32,
    )
    o_ref[...] = x_ref[...] + emb_tile[:, None, :]


@jax.jit
def kernel(x, g_id, embedding):
    batch = x.shape[0]
    num_blocks = batch // TB
    gid3 = g_id.astype(jnp.int32).reshape(num_blocks, 1, TB)
    return pl.pallas_call(
        _body,
        grid=(num_blocks,),
        in_specs=[
            pl.BlockSpec((1, 1, TB), lambda i: (i, 0, 0)),
            pl.BlockSpec((TB, HIST, HIDDEN), lambda i: (i, 0, 0)),
            pl.BlockSpec((GRID_NUNQ, HIDDEN), lambda i: (0, 0)),
        ],
        out_specs=pl.BlockSpec((TB, HIST, HIDDEN), lambda i: (i, 0, 0)),
        out_shape=jax.ShapeDtypeStruct((batch, HIST, HIDDEN), jnp.float32),
        compiler_params=pltpu.CompilerParams(
            dimension_semantics=("arbitrary",),
        ),
    )(gid3, x, embedding)


# speedup vs baseline: 1.0654x; 1.0057x over previous
import functools

import jax
import jax.numpy as jnp
from jax.experimental import pallas as pl
from jax.experimental.pallas import tpu as pltpu

GRID_NUNQ = 384
HIDDEN = 128
HIST = 200
TB = 128


def _body(gid_ref, x_ref, emb_ref, o_ref):
    ids = gid_ref[0]
    one_hot = (
        jax.lax.broadcasted_iota(jnp.int32, (GRID_NUNQ, TB), 0) == ids
    ).astype(jnp.float32)
    emb_tile = jax.lax.dot_general(
        one_hot,
        emb_ref[...],
        (((0,), (0,)), ((), ())),
        preferred_element_type=jnp.float32,
    )
    o_ref[...] = x_ref[...] + emb_tile[:, None, :]


@jax.jit
def kernel(x, g_id, embedding):
    batch = x.shape[0]
    num_blocks = batch // TB
    gid3 = g_id.astype(jnp.int32).reshape(num_blocks, 1, TB)
    return pl.pallas_call(
        _body,
        grid=(num_blocks,),
        in_specs=[
            pl.BlockSpec((1, 1, TB), lambda i: (i, 0, 0)),
            pl.BlockSpec((TB, HIST, HIDDEN), lambda i: (i, 0, 0)),
            pl.BlockSpec((GRID_NUNQ, HIDDEN), lambda i: (0, 0)),
        ],
        out_specs=pl.BlockSpec((TB, HIST, HIDDEN), lambda i: (i, 0, 0)),
        out_shape=jax.ShapeDtypeStruct((batch, HIST, HIDDEN), jnp.float32),
        compiler_params=pltpu.CompilerParams(
            dimension_semantics=("arbitrary",),
        ),
    )(gid3, x, embedding)
